# baseline (speedup 1.0000x reference)
import functools

import jax
import jax.numpy as jnp
from jax import lax
from jax.experimental import pallas as pl
from jax.experimental.pallas import tpu as pltpu
from jax.experimental.pallas import tpu_sc as plsc

NUM_FIELDS = 26
EMBED_DIM = 32
BATCH = 16384

_info = plsc.get_sparse_core_info()
_NC, _NS = _info.num_cores, _info.num_subcores
_NW = _NC * _NS
_BPW = BATCH // _NW


def _sc_body(*refs):
    fs = refs[:NUM_FIELDS]
    Ws = refs[NUM_FIELDS:2 * NUM_FIELDS]
    out = refs[2 * NUM_FIELDS]
    idx_v = refs[2 * NUM_FIELDS + 1]
    rows_v = refs[2 * NUM_FIELDS + 2]
    sem = refs[2 * NUM_FIELDS + 3]

    wid = lax.axis_index("s") * _NC + lax.axis_index("c")
    base = wid * _BPW
    for i in range(NUM_FIELDS):
        pltpu.sync_copy(fs[i].at[pl.ds(base, _BPW)], idx_v)
        pltpu.async_copy(Ws[i].at[idx_v], rows_v, sem).wait()
        pltpu.sync_copy(rows_v, out.at[pl.ds(base, _BPW), i])


@jax.jit
def _cat_embed(*args):
    mesh = plsc.VectorSubcoreMesh(core_axis_name="c", subcore_axis_name="s")
    k = functools.partial(
        pl.kernel,
        mesh=mesh,
        out_type=jax.ShapeDtypeStruct((BATCH, NUM_FIELDS, EMBED_DIM),
                                      jnp.float32),
        scratch_types=[
            pltpu.VMEM((_BPW,), jnp.int32),
            pltpu.VMEM((_BPW, EMBED_DIM), jnp.float32),
            pltpu.SemaphoreType.DMA,
        ],
        compiler_params=pltpu.CompilerParams(use_tc_tiling_on_sc=False),
    )(_sc_body)
    out3 = k(*args)
    return out3.reshape(BATCH, NUM_FIELDS * EMBED_DIM)


def kernel(f0, f1, f2, f3, f4, f5, f6, f7, f8, f9, f10, f11, f12, f13,
           f14, f15, f16, f17, f18, f19, f20, f21, f22, f23, f24, f25,
           W0, W1, W2, W3, W4, W5, W6, W7, W8, W9, W10, W11, W12, W13,
           W14, W15, W16, W17, W18, W19, W20, W21, W22, W23, W24, W25):
    fields = [f0, f1, f2, f3, f4, f5, f6, f7, f8, f9, f10, f11, f12, f13,
              f14, f15, f16, f17, f18, f19, f20, f21, f22, f23, f24, f25]
    tables = [W0, W1, W2, W3, W4, W5, W6, W7, W8, W9, W10, W11, W12, W13,
              W14, W15, W16, W17, W18, W19, W20, W21, W22, W23, W24, W25]
    fields = [jnp.asarray(f, jnp.int32) for f in fields]
    return _cat_embed(*fields, *tables)


# speedup vs baseline: 1.0268x; 1.0268x over previous
import functools

import jax
import jax.numpy as jnp
from jax import lax
from jax.experimental import pallas as pl
from jax.experimental.pallas import tpu as pltpu
from jax.experimental.pallas import tpu_sc as plsc

NUM_FIELDS = 26
EMBED_DIM = 32
BATCH = 16384

_info = plsc.get_sparse_core_info()
_NC, _NS = _info.num_cores, _info.num_subcores
_NW = _NC * _NS
_BPW = BATCH // _NW


_NBUF = 6


def _sc_body(*refs):
    fs = refs[:NUM_FIELDS]
    Ws = refs[NUM_FIELDS:2 * NUM_FIELDS]
    out = refs[2 * NUM_FIELDS]
    r = 2 * NUM_FIELDS + 1
    idx_all = refs[r]
    bufs = refs[r + 1:r + 1 + _NBUF]
    isem = refs[r + 1 + _NBUF]
    gsems = refs[r + 2 + _NBUF:r + 2 + _NBUF + _NBUF]
    ssems = refs[r + 2 + 2 * _NBUF:r + 2 + 3 * _NBUF]

    wid = lax.axis_index("s") * _NC + lax.axis_index("c")
    base = wid * _BPW

    idescs = [pltpu.async_copy(fs[i].at[pl.ds(base, _BPW)], idx_all.at[i],
                               isem) for i in range(NUM_FIELDS)]
    for d in idescs:
        d.wait()

    gd = {}
    sd = {}
    for i in range(_NBUF):
        gd[i] = pltpu.async_copy(Ws[i].at[idx_all.at[i]], bufs[i], gsems[i])
    for i in range(NUM_FIELDS):
        s = i % _NBUF
        gd[i].wait()
        sd[i] = pltpu.async_copy(bufs[s], out.at[pl.ds(base, _BPW), i],
                                 ssems[s])
        j = i + _NBUF
        if j < NUM_FIELDS:
            sd[j - _NBUF].wait()
            gd[j] = pltpu.async_copy(Ws[j].at[idx_all.at[j]], bufs[s],
                                     gsems[s])
    for i in range(NUM_FIELDS - _NBUF, NUM_FIELDS):
        sd[i].wait()


@jax.jit
def _cat_embed(*args):
    mesh = plsc.VectorSubcoreMesh(core_axis_name="c", subcore_axis_name="s")
    k = functools.partial(
        pl.kernel,
        mesh=mesh,
        out_type=jax.ShapeDtypeStruct((BATCH, NUM_FIELDS, EMBED_DIM),
                                      jnp.float32),
        scratch_types=(
            [pltpu.VMEM((NUM_FIELDS, _BPW), jnp.int32)]
            + [pltpu.VMEM((_BPW, EMBED_DIM), jnp.float32)
               for _ in range(_NBUF)]
            + [pltpu.SemaphoreType.DMA for _ in range(1 + 2 * _NBUF)]
        ),
        compiler_params=pltpu.CompilerParams(use_tc_tiling_on_sc=False),
    )(_sc_body)
    out3 = k(*args)
    return out3.reshape(BATCH, NUM_FIELDS * EMBED_DIM)


def kernel(f0, f1, f2, f3, f4, f5, f6, f7, f8, f9, f10, f11, f12, f13,
           f14, f15, f16, f17, f18, f19, f20, f21, f22, f23, f24, f25,
           W0, W1, W2, W3, W4, W5, W6, W7, W8, W9, W10, W11, W12, W13,
           W14, W15, W16, W17, W18, W19, W20, W21, W22, W23, W24, W25):
    fields = [f0, f1, f2, f3, f4, f5, f6, f7, f8, f9, f10, f11, f12, f13,
              f14, f15, f16, f17, f18, f19, f20, f21, f22, f23, f24, f25]
    tables = [W0, W1, W2, W3, W4, W5, W6, W7, W8, W9, W10, W11, W12, W13,
              W14, W15, W16, W17, W18, W19, W20, W21, W22, W23, W24, W25]
    fields = [jnp.asarray(f, jnp.int32) for f in fields]
    return _cat_embed(*fields, *tables)


# speedup vs baseline: 1.6580x; 1.6147x over previous
import functools

import jax
import jax.numpy as jnp
from jax import lax
from jax.experimental import pallas as pl
from jax.experimental.pallas import tpu as pltpu
from jax.experimental.pallas import tpu_sc as plsc

NUM_FIELDS = 26
EMBED_DIM = 32
BATCH = 16384
VOCAB = 100000

_info = plsc.get_sparse_core_info()
_NC, _NS = _info.num_cores, _info.num_subcores
_NW = _NC * _NS

_CH = 8192


def _sc_body(*refs):
    fs = refs[:NUM_FIELDS]
    Ws = refs[NUM_FIELDS:2 * NUM_FIELDS]
    out = refs[2 * NUM_FIELDS]
    r = 2 * NUM_FIELDS + 1
    row_v, idx_v, out_v, rsem, isem, osem = refs[r:r + 6]

    wid = lax.axis_index("s") * _NC + lax.axis_index("c")

    prev_store = None
    for f in range(NUM_FIELDS):
        rd = pltpu.async_copy(Ws[f].at[wid], row_v, rsem)
        for c in range(BATCH // _CH):
            idm = pltpu.async_copy(fs[f].at[pl.ds(c * _CH, _CH)], idx_v, isem)
            idm.wait()
            if c == 0:
                rd.wait()
            if prev_store is not None:
                prev_store.wait()

            def gather_step(i, _):
                b = i * 128
                for u in range(8):
                    iv = idx_v[pl.ds(b + u * 16, 16)]
                    out_v[pl.ds(b + u * 16, 16)] = plsc.load_gather(
                        row_v, [iv])
                return 0

            lax.fori_loop(0, _CH // 128, gather_step, 0)
            prev_store = pltpu.async_copy(
                out_v, out.at[f * EMBED_DIM + wid, pl.ds(c * _CH, _CH)], osem)
    prev_store.wait()


@jax.jit
def _cat_embed(*args):
    mesh = plsc.VectorSubcoreMesh(core_axis_name="c", subcore_axis_name="s")
    k = functools.partial(
        pl.kernel,
        mesh=mesh,
        out_type=jax.ShapeDtypeStruct((NUM_FIELDS * EMBED_DIM, BATCH),
                                      jnp.float32),
        scratch_types=[
            pltpu.VMEM((VOCAB,), jnp.float32),
            pltpu.VMEM((_CH,), jnp.int32),
            pltpu.VMEM((_CH,), jnp.float32),
            pltpu.SemaphoreType.DMA,
            pltpu.SemaphoreType.DMA,
            pltpu.SemaphoreType.DMA,
        ],
        compiler_params=pltpu.CompilerParams(use_tc_tiling_on_sc=False,
                                             needs_layout_passes=False),
    )(_sc_body)
    out_t = k(*args)
    return out_t.T


def kernel(f0, f1, f2, f3, f4, f5, f6, f7, f8, f9, f10, f11, f12, f13,
           f14, f15, f16, f17, f18, f19, f20, f21, f22, f23, f24, f25,
           W0, W1, W2, W3, W4, W5, W6, W7, W8, W9, W10, W11, W12, W13,
           W14, W15, W16, W17, W18, W19, W20, W21, W22, W23, W24, W25):
    fields = [f0, f1, f2, f3, f4, f5, f6, f7, f8, f9, f10, f11, f12, f13,
              f14, f15, f16, f17, f18, f19, f20, f21, f22, f23, f24, f25]
    tables = [W0, W1, W2, W3, W4, W5, W6, W7, W8, W9, W10, W11, W12, W13,
              W14, W15, W16, W17, W18, W19, W20, W21, W22, W23, W24, W25]
    fields = [jnp.asarray(f, jnp.int32) for f in fields]
    tables_t = [jnp.transpose(W) for W in tables]
    return _cat_embed(*fields, *tables_t)


# speedup vs baseline: 4.3819x; 2.6429x over previous
import functools

import jax
import jax.numpy as jnp
from jax import lax
from jax.experimental import pallas as pl
from jax.experimental.pallas import tpu as pltpu
from jax.experimental.pallas import tpu_sc as plsc

NUM_FIELDS = 26
EMBED_DIM = 32
BATCH = 16384
VOCAB = 100000

_info = plsc.get_sparse_core_info()
_NC, _NS = _info.num_cores, _info.num_subcores
_NW = _NC * _NS

_CH = 8192


def _sc_body(*refs):
    fs = refs[:NUM_FIELDS]
    Ws = refs[NUM_FIELDS:2 * NUM_FIELDS]
    out = refs[2 * NUM_FIELDS]
    r = 2 * NUM_FIELDS + 1
    row_v, idx_v, out_v, rsem, isem, osem = refs[r:r + 6]

    wid = lax.axis_index("s") * _NC + lax.axis_index("c")

    prev_store = None
    for f in range(NUM_FIELDS):
        rd = pltpu.async_copy(Ws[f].at[wid], row_v, rsem)
        for c in range(BATCH // _CH):
            idm = pltpu.async_copy(fs[f].at[pl.ds(c * _CH, _CH)], idx_v, isem)
            idm.wait()
            if c == 0:
                rd.wait()
            if prev_store is not None:
                prev_store.wait()

            def gather_step(i, _):
                b = i * 128
                for u in range(8):
                    iv = idx_v[pl.ds(b + u * 16, 16)]
                    out_v[pl.ds(b + u * 16, 16)] = plsc.load_gather(
                        row_v, [iv])
                return 0

            lax.fori_loop(0, _CH // 128, gather_step, 0)
            prev_store = pltpu.async_copy(
                out_v, out.at[f * EMBED_DIM + wid, pl.ds(c * _CH, _CH)], osem)
    prev_store.wait()


@jax.jit
def _cat_embed(*args):
    mesh = plsc.VectorSubcoreMesh(core_axis_name="c", subcore_axis_name="s")
    k = functools.partial(
        pl.kernel,
        mesh=mesh,
        out_type=jax.ShapeDtypeStruct((NUM_FIELDS * EMBED_DIM, BATCH),
                                      jnp.float32),
        scratch_types=[
            pltpu.VMEM((VOCAB,), jnp.float32),
            pltpu.VMEM((_CH,), jnp.int32),
            pltpu.VMEM((_CH,), jnp.float32),
            pltpu.SemaphoreType.DMA,
            pltpu.SemaphoreType.DMA,
            pltpu.SemaphoreType.DMA,
        ],
        compiler_params=pltpu.CompilerParams(use_tc_tiling_on_sc=True,
                                             needs_layout_passes=False),
    )(_sc_body)
    out_t = k(*args)
    return out_t.T


def kernel(f0, f1, f2, f3, f4, f5, f6, f7, f8, f9, f10, f11, f12, f13,
           f14, f15, f16, f17, f18, f19, f20, f21, f22, f23, f24, f25,
           W0, W1, W2, W3, W4, W5, W6, W7, W8, W9, W10, W11, W12, W13,
           W14, W15, W16, W17, W18, W19, W20, W21, W22, W23, W24, W25):
    fields = [f0, f1, f2, f3, f4, f5, f6, f7, f8, f9, f10, f11, f12, f13,
              f14, f15, f16, f17, f18, f19, f20, f21, f22, f23, f24, f25]
    tables = [W0, W1, W2, W3, W4, W5, W6, W7, W8, W9, W10, W11, W12, W13,
              W14, W15, W16, W17, W18, W19, W20, W21, W22, W23, W24, W25]
    fields = [jnp.asarray(f, jnp.int32) for f in fields]
    tables_t = [jnp.transpose(W) for W in tables]
    return _cat_embed(*fields, *tables_t)
